# baseline (speedup 1.0000x reference)
import functools

import jax
import jax.numpy as jnp
from jax import lax
from jax.experimental import pallas as pl
from jax.experimental.pallas import tpu as pltpu
from jax.experimental.pallas import tpu_sc as plsc

_L = 16
_TILE = 128
_CHUNK = 128
_CAP = 16448
_BLK = 64
_HI = _BLK - _L


def _iota():
    return lax.iota(jnp.int32, _L)


@functools.lru_cache(maxsize=None)
def _build_phase1(B, F, N):
    info = plsc.get_sparse_core_info()
    NC, NS = info.num_cores, info.num_subcores
    NW = NC * NS
    NT = -(-N // _TILE)
    per = NT // NW
    extra = NT - per * NW
    SB = B + 8
    n_groups = B // _L

    mesh = plsc.VectorSubcoreMesh(core_axis_name="c", subcore_axis_name="s")

    @functools.partial(
        pl.kernel,
        mesh=mesh,
        out_type=(
            jax.ShapeDtypeStruct((SB, _TILE), jnp.float32),
            jax.ShapeDtypeStruct((SB, _TILE), jnp.float32),
        ),
        compiler_params=pltpu.CompilerParams(
            needs_layout_passes=False, use_tc_tiling_on_sc=True
        ),
        scratch_types=[
            pltpu.VMEM((_CAP,), jnp.int32),
            pltpu.VMEM((_CAP,), jnp.int32),
            pltpu.VMEM((_CAP,), jnp.int32),
            pltpu.VMEM((_CAP,), jnp.int32),
            pltpu.VMEM((32,), jnp.int32),
            pltpu.VMEM((256,), jnp.int32),
            pltpu.VMEM((256,), jnp.int32),
            pltpu.VMEM((256,), jnp.int32),
            pltpu.VMEM((4, F, _TILE), jnp.float32),
            pltpu.VMEM((2, _BLK, _TILE), jnp.float32),
            pltpu.VMEM((2, _BLK), jnp.int32),
            pltpu.SemaphoreType.DMA,
            pltpu.SemaphoreType.DMA,
            pltpu.SemaphoreType.DMA,
            pltpu.SemaphoreType.DMA,
            pltpu.SemaphoreType.DMA,
        ],
    )
    def k(uT_h, iT_h, uidx_h, iidx_h, urows_h, irows_h,
          A, UL, BL, D, segv, hist, utl, utc, slab, blk, bix,
          sem0, sem1, sem2, sem3, semS):
        wid = lax.axis_index("s") * NC + lax.axis_index("c")
        lanes = _iota()
        base_ut = wid * per + jnp.minimum(wid, extra)
        n_ut = per + (wid < extra).astype(jnp.int32)
        lo_u = base_ut * _TILE
        hi_u = (base_ut + n_ut) * _TILE
        ones = jnp.ones((_L,), jnp.int32)

        def one_table(tab_h, idx_h, rows_h):
            for g in range(256 // _L):
                hist[pl.ds(g * _L, _L)] = jnp.zeros((_L,), jnp.int32)
            for p in range(2):
                for g in range(_BLK // _L):
                    bix[p, pl.ds(g * _L, _L)] = jnp.full((_L,), B, jnp.int32)
            pltpu.sync_copy(idx_h, A.at[pl.ds(0, B)])

            def scan_g(g, cnt):
                u = A[pl.ds(g * _L, _L)]
                b = g * _L + lanes
                m = (u >= lo_u) & (u < hi_u)
                pos = cnt + plsc.cumsum(m.astype(jnp.int32)) - 1
                plsc.store_scatter(UL, [pos], u, mask=m)
                plsc.store_scatter(BL, [pos], b, mask=m)
                ut_rel = lax.shift_right_logical(u, 7) - base_ut
                plsc.addupdate_scatter(
                    hist, [jnp.where(m, ut_rel, 255)], ones, mask=m)
                return cnt + plsc.all_reduce_population_count(m)[0]

            cnt = lax.fori_loop(0, n_groups, scan_g, 0)

            def comp_g(g, c2):
                ids = g * _L + lanes
                h = hist[pl.ds(g * _L, _L)]
                m2 = (h > 0) & (ids < n_ut)
                pos = c2 + plsc.cumsum(m2.astype(jnp.int32)) - 1
                plsc.store_scatter(utl, [pos], ids, mask=m2)
                plsc.store_scatter(utc, [pos], h, mask=m2)
                return c2 + plsc.all_reduce_population_count(m2)[0]

            n_active = lax.fori_loop(0, 256 // _L, comp_g, 0)

            list_groups = lax.shift_right_logical(cnt + _L - 1, 4)
            seg_start = []
            seg_len = []
            st2 = 0
            for s in range(8):
                seg_start.append(st2)

                def bin_g(g, c3, s=s):
                    u = UL[pl.ds(g * _L, _L)]
                    b = BL[pl.ds(g * _L, _L)]
                    m = lax.shift_right_logical(u - lo_u, 12) == s
                    m = m & (g * _L + lanes < cnt)
                    pos = c3 + plsc.cumsum(m.astype(jnp.int32)) - 1
                    plsc.store_scatter(A, [pos], u, mask=m)
                    plsc.store_scatter(D, [pos], b, mask=m)
                    return c3 + plsc.all_reduce_population_count(m)[0]

                st2 = lax.fori_loop(0, list_groups, bin_g, st2)
                seg_len.append(st2 - seg_start[s])
            segs_v = jnp.zeros((_L,), jnp.int32)
            lens_v = jnp.zeros((_L,), jnp.int32)
            for s in range(8):
                segs_v = jnp.where(lanes == s, seg_start[s], segs_v)
                lens_v = jnp.where(lanes == s, seg_len[s], lens_v)
            segv[pl.ds(0, _L)] = segs_v
            segv[pl.ds(_L, _L)] = lens_v

            sems = (sem0, sem1, sem2, sem3)

            def fetch(j, ring):
                ut_rel = utl[pl.ds(j, _L)][0]
                u0 = (base_ut + ut_rel) * _TILE
                for r in range(4):
                    @pl.when(ring == r)
                    def _(r=r):
                        for ft in range(F // 8):
                            pltpu.async_copy(
                                tab_h.at[pl.ds(ft * 8, 8), pl.ds(u0, _TILE)],
                                slab.at[r, pl.ds(ft * 8, 8)], sems[r])

            for jj in range(3):
                @pl.when(jj < n_active)
                def _(jj=jj):
                    fetch(jj, jj)

            def flush(fn):
                for p in range(2):
                    @pl.when(jnp.bitwise_and(fn, 1) == p)
                    def _(p=p):
                        pltpu.async_copy(
                            blk.at[p], rows_h.at[bix.at[p]], semS)

                @pl.when(fn >= 1)
                def _():
                    pltpu.make_async_copy(
                        rows_h.at[pl.ds(0, _BLK)], blk.at[0], semS).wait()

            def refill_bix(fn):
                for p in range(2):
                    @pl.when(jnp.bitwise_and(fn, 1) == p)
                    def _(p=p):
                        for g in range(_BLK // _L):
                            bix[p, pl.ds(g * _L, _L)] = jnp.full(
                                (_L,), B, jnp.int32)

            def ut_loop(j, carry):
                fc, fn = carry
                ring = jnp.bitwise_and(j, 3)

                @pl.when(j + 3 < n_active)
                def _():
                    fetch(j + 3, jnp.bitwise_and(j + 3, 3))

                for r in range(4):
                    @pl.when(ring == r)
                    def _(r=r):
                        pltpu.make_async_copy(
                            tab_h.at[pl.ds(0, F), pl.ds(0, _TILE)],
                            slab.at[r], sems[r]).wait()

                ut_rel = utl[pl.ds(j, _L)][0]
                k_ut = utc[pl.ds(j, _L)][0]
                s_id = lax.shift_right_logical(ut_rel, 5)
                seg0 = segv[pl.ds(s_id, _L)][0]
                slen = segv[pl.ds(s_id + _L, _L)][0]
                g0 = lax.shift_right_logical(seg0, 4)
                g1 = lax.shift_right_logical(seg0 + slen + _L - 1, 4)

                def rescan(g, st):
                    u = A[pl.ds(g * _L, _L)]
                    b = D[pl.ds(g * _L, _L)]
                    e = g * _L + lanes
                    m = (lax.shift_right_logical(u, 7) - base_ut == ut_rel)
                    m = m & (e >= seg0) & (e < seg0 + slen)
                    pos = st + plsc.cumsum(m.astype(jnp.int32)) - 1
                    plsc.store_scatter(UL, [pos], u, mask=m)
                    plsc.store_scatter(BL, [pos], b, mask=m)
                    return st + plsc.all_reduce_population_count(m)[0]

                lax.fori_loop(g0, g1, rescan, 0)

                n_chunks = lax.shift_right_logical(k_ut + _L - 1, 4)
                ringv = jnp.full((_L,), ring, jnp.int32)

                def ext(ci, c):
                    fc, fn = c
                    uvec = UL[pl.ds(ci * _L, _L)]
                    bvec = BL[pl.ds(ci * _L, _L)]
                    valid = ci * _L + lanes < k_ut
                    ui = jnp.bitwise_and(uvec, _TILE - 1)
                    par = jnp.bitwise_and(fn, 1)
                    parv = jnp.full((_L,), par, jnp.int32)
                    nsl = plsc.cumsum(valid.astype(jnp.int32))
                    slotv = fc + nsl - 1
                    plsc.store_scatter(bix, [parv, slotv], bvec, mask=valid)

                    for f in range(F):
                        fv = jnp.full((_L,), f, jnp.int32)
                        vals = plsc.load_gather(slab, [ringv, fv, ui])
                        plsc.store_scatter(
                            blk, [parv, slotv, fv], vals, mask=valid)
                    fc2 = fc + plsc.all_reduce_population_count(valid)[0]
                    do_flush = fc2 > _HI

                    @pl.when(do_flush)
                    def _():
                        flush(fn)
                        refill_bix(fn + 1)

                    fc3 = jnp.where(do_flush, 0, fc2)
                    fn2 = fn + do_flush.astype(jnp.int32)
                    return (fc3, fn2)

                return lax.fori_loop(0, n_chunks, ext, (fc, fn))

            fc, fn = lax.fori_loop(0, n_active, ut_loop, (0, 0))

            @pl.when(fc > 0)
            def _():
                flush(fn)

            fn_tot = fn + (fc > 0).astype(jnp.int32)

            @pl.when(fn_tot >= 1)
            def _():
                pltpu.make_async_copy(
                    rows_h.at[pl.ds(0, _BLK)], blk.at[0], semS).wait()

        one_table(uT_h, uidx_h, urows_h)
        one_table(iT_h, iidx_h, irows_h)

    return k


@functools.lru_cache(maxsize=None)
def _build_phase2(B, F, SB):
    info = plsc.get_sparse_core_info()
    NC, NS = info.num_cores, info.num_subcores
    NW = NC * NS
    b_per_w = B // NW
    half = b_per_w // 2
    n_chunks = b_per_w // _CHUNK

    mesh = plsc.VectorSubcoreMesh(core_axis_name="c", subcore_axis_name="s")

    @functools.partial(
        pl.kernel,
        mesh=mesh,
        out_type=jax.ShapeDtypeStruct((B,), jnp.float32),
        compiler_params=pltpu.CompilerParams(
            needs_layout_passes=False, use_tc_tiling_on_sc=False
        ),
        scratch_types=[
            pltpu.VMEM((half, _TILE), jnp.float32),
            pltpu.VMEM((half, _TILE), jnp.float32),
            pltpu.VMEM((b_per_w,), jnp.int32),
            pltpu.VMEM((b_per_w,), jnp.int32),
            pltpu.VMEM((b_per_w,), jnp.float32),
            pltpu.VMEM((b_per_w,), jnp.float32),
            pltpu.VMEM((b_per_w,), jnp.float32),
            pltpu.SemaphoreType.DMA,
        ],
    )
    def k(urows_h, irows_h, ub_h, ib_h, uidx_h, iidx_h, out_h,
          uv, iv, uidx_v, iidx_v, ubv, ibv, outv, sem):
        wid = lax.axis_index("s") * NC + lax.axis_index("c")
        lanes = _iota()
        base = wid * b_per_w
        pltpu.sync_copy(uidx_h.at[pl.ds(base, b_per_w)], uidx_v)
        pltpu.sync_copy(iidx_h.at[pl.ds(base, b_per_w)], iidx_v)
        for c in range(n_chunks):
            s = pl.ds(c * _CHUNK, _CHUNK)
            pltpu.async_copy(ub_h.at[uidx_v.at[s]], ubv.at[s], sem)
            pltpu.async_copy(ib_h.at[iidx_v.at[s]], ibv.at[s], sem)

        for h in range(2):
            pltpu.sync_copy(urows_h.at[pl.ds(base + h * half, half)], uv)
            pltpu.sync_copy(irows_h.at[pl.ds(base + h * half, half)], iv)

            def group(g, carry):
                rows = g * _L + lanes
                acc = jnp.zeros((_L,), jnp.float32)
                for f in range(F):
                    cols = jnp.bitwise_and(f + lanes, F - 1)
                    ug = plsc.load_gather(uv, [rows, cols])
                    ig = plsc.load_gather(iv, [rows, cols])
                    acc = acc + ug * ig
                outv[pl.ds(h * half + g * _L, _L)] = acc
                return carry

            lax.fori_loop(0, half // _L, group, 0)

        pltpu.make_async_copy(ub_h.at[pl.ds(0, b_per_w)], ubv, sem).wait()
        pltpu.make_async_copy(ib_h.at[pl.ds(0, b_per_w)], ibv, sem).wait()

        def addb(g, carry):
            s = pl.ds(g * _L, _L)
            outv[s] = outv[s] + ubv[s] + ibv[s]
            return carry

        lax.fori_loop(0, b_per_w // _L, addb, 0)
        pltpu.sync_copy(outv, out_h.at[pl.ds(base, b_per_w)])

    return k


def kernel(u_emb, i_emb, u_bias, i_bias, u_idx, i_idx):
    B = u_idx.shape[0]
    N, F = u_emb.shape
    u32 = u_idx.astype(jnp.int32)
    i32 = i_idx.astype(jnp.int32)
    urows, irows = _build_phase1(B, F, N)(u_emb.T, i_emb.T, u32, i32)
    return _build_phase2(B, F, B + 8)(
        urows, irows, u_bias.reshape(-1), i_bias.reshape(-1), u32, i32
    )


# speedup vs baseline: 1.2230x; 1.2230x over previous
import functools

import jax
import jax.numpy as jnp
from jax import lax
from jax.experimental import pallas as pl
from jax.experimental.pallas import tpu as pltpu
from jax.experimental.pallas import tpu_sc as plsc

_L = 16
_TILE = 128
_CHUNK = 128
_CAP = 16448
_BLK = 112
_HI = _BLK - _L


def _iota():
    return lax.iota(jnp.int32, _L)


@functools.lru_cache(maxsize=None)
def _build_phase1(B, F, N):
    info = plsc.get_sparse_core_info()
    NC, NS = info.num_cores, info.num_subcores
    NW = NC * NS
    NT = -(-N // _TILE)
    per = NT // NW
    extra = NT - per * NW
    SB = B + 8
    n_groups = B // _L

    mesh = plsc.VectorSubcoreMesh(core_axis_name="c", subcore_axis_name="s")

    @functools.partial(
        pl.kernel,
        mesh=mesh,
        out_type=(
            jax.ShapeDtypeStruct((SB, _TILE), jnp.float32),
            jax.ShapeDtypeStruct((SB, _TILE), jnp.float32),
        ),
        compiler_params=pltpu.CompilerParams(
            needs_layout_passes=False, use_tc_tiling_on_sc=True
        ),
        scratch_types=[
            pltpu.VMEM((_CAP,), jnp.int32),
            pltpu.VMEM((_CAP,), jnp.int32),
            pltpu.VMEM((_CAP,), jnp.int32),
            pltpu.VMEM((_CAP,), jnp.int32),
            pltpu.VMEM((32,), jnp.int32),
            pltpu.VMEM((256,), jnp.int32),
            pltpu.VMEM((256,), jnp.int32),
            pltpu.VMEM((256,), jnp.int32),
            pltpu.VMEM((4, F, _TILE), jnp.float32),
            pltpu.VMEM((2, _BLK, _TILE), jnp.float32),
            pltpu.VMEM((2, _BLK), jnp.int32),
            pltpu.SemaphoreType.DMA,
            pltpu.SemaphoreType.DMA,
            pltpu.SemaphoreType.DMA,
            pltpu.SemaphoreType.DMA,
            pltpu.SemaphoreType.DMA,
        ],
    )
    def k(uT_h, iT_h, uidx_h, iidx_h, urows_h, irows_h,
          A, UL, BL, D, segv, hist, utl, utc, slab, blk, bix,
          sem0, sem1, sem2, sem3, semS):
        wid = lax.axis_index("s") * NC + lax.axis_index("c")
        lanes = _iota()
        base_ut = wid * per + jnp.minimum(wid, extra)
        n_ut = per + (wid < extra).astype(jnp.int32)
        lo_u = base_ut * _TILE
        hi_u = (base_ut + n_ut) * _TILE
        ones = jnp.ones((_L,), jnp.int32)

        def one_table(tab_h, idx_h, rows_h):
            for g in range(256 // _L):
                hist[pl.ds(g * _L, _L)] = jnp.zeros((_L,), jnp.int32)
            for p in range(2):
                for g in range(_BLK // _L):
                    bix[p, pl.ds(g * _L, _L)] = jnp.full((_L,), B, jnp.int32)
            pltpu.sync_copy(idx_h, A.at[pl.ds(0, B)])

            def scan_g(g, cnt):
                u = A[pl.ds(g * _L, _L)]
                b = g * _L + lanes
                m = (u >= lo_u) & (u < hi_u)
                pos = cnt + plsc.cumsum(m.astype(jnp.int32)) - 1
                plsc.store_scatter(UL, [pos], u, mask=m)
                plsc.store_scatter(BL, [pos], b, mask=m)
                ut_rel = lax.shift_right_logical(u, 7) - base_ut
                plsc.addupdate_scatter(
                    hist, [jnp.where(m, ut_rel, 255)], ones, mask=m)
                return cnt + plsc.all_reduce_population_count(m)[0]

            cnt = lax.fori_loop(0, n_groups, scan_g, 0)

            def comp_g(g, c2):
                ids = g * _L + lanes
                h = hist[pl.ds(g * _L, _L)]
                m2 = (h > 0) & (ids < n_ut)
                pos = c2 + plsc.cumsum(m2.astype(jnp.int32)) - 1
                plsc.store_scatter(utl, [pos], ids, mask=m2)
                plsc.store_scatter(utc, [pos], h, mask=m2)
                return c2 + plsc.all_reduce_population_count(m2)[0]

            n_active = lax.fori_loop(0, 256 // _L, comp_g, 0)

            list_groups = lax.shift_right_logical(cnt + _L - 1, 4)
            seg_start = []
            seg_len = []
            st2 = 0
            for s in range(8):
                seg_start.append(st2)

                def bin_g(g, c3, s=s):
                    u = UL[pl.ds(g * _L, _L)]
                    b = BL[pl.ds(g * _L, _L)]
                    m = lax.shift_right_logical(u - lo_u, 12) == s
                    m = m & (g * _L + lanes < cnt)
                    pos = c3 + plsc.cumsum(m.astype(jnp.int32)) - 1
                    plsc.store_scatter(A, [pos], u, mask=m)
                    plsc.store_scatter(D, [pos], b, mask=m)
                    return c3 + plsc.all_reduce_population_count(m)[0]

                st2 = lax.fori_loop(0, list_groups, bin_g, st2)
                seg_len.append(st2 - seg_start[s])
            segs_v = jnp.zeros((_L,), jnp.int32)
            lens_v = jnp.zeros((_L,), jnp.int32)
            for s in range(8):
                segs_v = jnp.where(lanes == s, seg_start[s], segs_v)
                lens_v = jnp.where(lanes == s, seg_len[s], lens_v)
            segv[pl.ds(0, _L)] = segs_v
            segv[pl.ds(_L, _L)] = lens_v

            sems = (sem0, sem1, sem2, sem3)

            def fetch(j, ring):
                ut_rel = utl[pl.ds(j, _L)][0]
                u0 = (base_ut + ut_rel) * _TILE
                for r in range(4):
                    @pl.when(ring == r)
                    def _(r=r):
                        for ft in range(F // 8):
                            pltpu.async_copy(
                                tab_h.at[pl.ds(ft * 8, 8), pl.ds(u0, _TILE)],
                                slab.at[r, pl.ds(ft * 8, 8)], sems[r])

            for jj in range(3):
                @pl.when(jj < n_active)
                def _(jj=jj):
                    fetch(jj, jj)

            def flush(fn):
                for p in range(2):
                    @pl.when(jnp.bitwise_and(fn, 1) == p)
                    def _(p=p):
                        pltpu.async_copy(
                            blk.at[p], rows_h.at[bix.at[p]], semS)

                @pl.when(fn >= 1)
                def _():
                    pltpu.make_async_copy(
                        rows_h.at[pl.ds(0, _BLK)], blk.at[0], semS).wait()

            def refill_bix(fn):
                for p in range(2):
                    @pl.when(jnp.bitwise_and(fn, 1) == p)
                    def _(p=p):
                        for g in range(_BLK // _L):
                            bix[p, pl.ds(g * _L, _L)] = jnp.full(
                                (_L,), B, jnp.int32)

            def ut_loop(j, carry):
                fc, fn = carry
                ring = jnp.bitwise_and(j, 3)

                @pl.when(j + 3 < n_active)
                def _():
                    fetch(j + 3, jnp.bitwise_and(j + 3, 3))

                for r in range(4):
                    @pl.when(ring == r)
                    def _(r=r):
                        pltpu.make_async_copy(
                            tab_h.at[pl.ds(0, F), pl.ds(0, _TILE)],
                            slab.at[r], sems[r]).wait()

                ut_rel = utl[pl.ds(j, _L)][0]
                k_ut = utc[pl.ds(j, _L)][0]
                s_id = lax.shift_right_logical(ut_rel, 5)
                seg0 = segv[pl.ds(s_id, _L)][0]
                slen = segv[pl.ds(s_id + _L, _L)][0]
                g0 = lax.shift_right_logical(seg0, 4)
                g1 = lax.shift_right_logical(seg0 + slen + _L - 1, 4)

                def rescan(g, st):
                    u = A[pl.ds(g * _L, _L)]
                    b = D[pl.ds(g * _L, _L)]
                    e = g * _L + lanes
                    m = (lax.shift_right_logical(u, 7) - base_ut == ut_rel)
                    m = m & (e >= seg0) & (e < seg0 + slen)
                    pos = st + plsc.cumsum(m.astype(jnp.int32)) - 1
                    plsc.store_scatter(UL, [pos], u, mask=m)
                    plsc.store_scatter(BL, [pos], b, mask=m)
                    return st + plsc.all_reduce_population_count(m)[0]

                lax.fori_loop(g0, g1, rescan, 0)

                n_chunks = lax.shift_right_logical(k_ut + _L - 1, 4)
                ringv = jnp.full((_L,), ring, jnp.int32)

                def ext(ci, c):
                    fc, fn = c
                    uvec = UL[pl.ds(ci * _L, _L)]
                    bvec = BL[pl.ds(ci * _L, _L)]
                    valid = ci * _L + lanes < k_ut
                    ui = jnp.bitwise_and(uvec, _TILE - 1)
                    par = jnp.bitwise_and(fn, 1)
                    parv = jnp.full((_L,), par, jnp.int32)
                    nsl = plsc.cumsum(valid.astype(jnp.int32))
                    slotv = fc + nsl - 1
                    plsc.store_scatter(bix, [parv, slotv], bvec, mask=valid)

                    for f in range(F):
                        fv = jnp.full((_L,), f, jnp.int32)
                        vals = plsc.load_gather(slab, [ringv, fv, ui])
                        plsc.store_scatter(
                            blk, [parv, slotv, fv], vals, mask=valid)
                    fc2 = fc + plsc.all_reduce_population_count(valid)[0]
                    do_flush = fc2 > _HI

                    @pl.when(do_flush)
                    def _():
                        flush(fn)
                        refill_bix(fn + 1)

                    fc3 = jnp.where(do_flush, 0, fc2)
                    fn2 = fn + do_flush.astype(jnp.int32)
                    return (fc3, fn2)

                return lax.fori_loop(0, n_chunks, ext, (fc, fn))

            fc, fn = lax.fori_loop(0, n_active, ut_loop, (0, 0))

            @pl.when(fc > 0)
            def _():
                flush(fn)

            fn_tot = fn + (fc > 0).astype(jnp.int32)

            @pl.when(fn_tot >= 1)
            def _():
                pltpu.make_async_copy(
                    rows_h.at[pl.ds(0, _BLK)], blk.at[0], semS).wait()

        one_table(uT_h, uidx_h, urows_h)
        one_table(iT_h, iidx_h, irows_h)

    return k


@functools.lru_cache(maxsize=None)
def _build_phase2(B, F, SB):
    info = plsc.get_sparse_core_info()
    NC, NS = info.num_cores, info.num_subcores
    NW = NC * NS
    b_per_w = B // NW
    half = b_per_w // 2
    n_chunks = b_per_w // _CHUNK

    mesh = plsc.VectorSubcoreMesh(core_axis_name="c", subcore_axis_name="s")

    @functools.partial(
        pl.kernel,
        mesh=mesh,
        out_type=jax.ShapeDtypeStruct((B,), jnp.float32),
        compiler_params=pltpu.CompilerParams(
            needs_layout_passes=False, use_tc_tiling_on_sc=False
        ),
        scratch_types=[
            pltpu.VMEM((half, _TILE), jnp.float32),
            pltpu.VMEM((half, _TILE), jnp.float32),
            pltpu.VMEM((b_per_w,), jnp.int32),
            pltpu.VMEM((b_per_w,), jnp.int32),
            pltpu.VMEM((b_per_w,), jnp.float32),
            pltpu.VMEM((b_per_w,), jnp.float32),
            pltpu.VMEM((b_per_w,), jnp.float32),
            pltpu.SemaphoreType.DMA,
        ],
    )
    def k(urows_h, irows_h, ub_h, ib_h, uidx_h, iidx_h, out_h,
          uv, iv, uidx_v, iidx_v, ubv, ibv, outv, sem):
        wid = lax.axis_index("s") * NC + lax.axis_index("c")
        lanes = _iota()
        base = wid * b_per_w
        pltpu.sync_copy(uidx_h.at[pl.ds(base, b_per_w)], uidx_v)
        pltpu.sync_copy(iidx_h.at[pl.ds(base, b_per_w)], iidx_v)
        for c in range(n_chunks):
            s = pl.ds(c * _CHUNK, _CHUNK)
            pltpu.async_copy(ub_h.at[uidx_v.at[s]], ubv.at[s], sem)
            pltpu.async_copy(ib_h.at[iidx_v.at[s]], ibv.at[s], sem)

        for h in range(2):
            pltpu.sync_copy(urows_h.at[pl.ds(base + h * half, half)], uv)
            pltpu.sync_copy(irows_h.at[pl.ds(base + h * half, half)], iv)

            def group(g, carry):
                rows = g * _L + lanes
                acc = jnp.zeros((_L,), jnp.float32)
                for f in range(F):
                    cols = jnp.bitwise_and(f + lanes, F - 1)
                    ug = plsc.load_gather(uv, [rows, cols])
                    ig = plsc.load_gather(iv, [rows, cols])
                    acc = acc + ug * ig
                outv[pl.ds(h * half + g * _L, _L)] = acc
                return carry

            lax.fori_loop(0, half // _L, group, 0)

        pltpu.make_async_copy(ub_h.at[pl.ds(0, b_per_w)], ubv, sem).wait()
        pltpu.make_async_copy(ib_h.at[pl.ds(0, b_per_w)], ibv, sem).wait()

        def addb(g, carry):
            s = pl.ds(g * _L, _L)
            outv[s] = outv[s] + ubv[s] + ibv[s]
            return carry

        lax.fori_loop(0, b_per_w // _L, addb, 0)
        pltpu.sync_copy(outv, out_h.at[pl.ds(base, b_per_w)])

    return k


def kernel(u_emb, i_emb, u_bias, i_bias, u_idx, i_idx):
    B = u_idx.shape[0]
    N, F = u_emb.shape
    u32 = u_idx.astype(jnp.int32)
    i32 = i_idx.astype(jnp.int32)
    urows, irows = _build_phase1(B, F, N)(u_emb.T, i_emb.T, u32, i32)
    return _build_phase2(B, F, B + 8)(
        urows, irows, u_bias.reshape(-1), i_bias.reshape(-1), u32, i32
    )


# speedup vs baseline: 1.2333x; 1.0084x over previous
import functools

import jax
import jax.numpy as jnp
from jax import lax
from jax.experimental import pallas as pl
from jax.experimental.pallas import tpu as pltpu
from jax.experimental.pallas import tpu_sc as plsc

_L = 16
_TILE = 128
_CHUNK = 128
_CAP = 16448
_BLK = 112
_HI = _BLK - _L


def _iota():
    return lax.iota(jnp.int32, _L)


@functools.lru_cache(maxsize=None)
def _build_phase1(B, F, N):
    info = plsc.get_sparse_core_info()
    NC, NS = info.num_cores, info.num_subcores
    NW = NC * NS
    NT = -(-N // _TILE)
    per = NT // NW
    extra = NT - per * NW
    SB = B + 8
    n_groups = B // _L

    mesh = plsc.VectorSubcoreMesh(core_axis_name="c", subcore_axis_name="s")

    @functools.partial(
        pl.kernel,
        mesh=mesh,
        out_type=(
            jax.ShapeDtypeStruct((SB, _TILE), jnp.float32),
            jax.ShapeDtypeStruct((SB, _TILE), jnp.float32),
        ),
        compiler_params=pltpu.CompilerParams(
            needs_layout_passes=False, use_tc_tiling_on_sc=True
        ),
        scratch_types=[
            pltpu.VMEM((_CAP,), jnp.int32),
            pltpu.VMEM((_CAP,), jnp.int32),
            pltpu.VMEM((_CAP,), jnp.int32),
            pltpu.VMEM((_CAP,), jnp.int32),
            pltpu.VMEM((32,), jnp.int32),
            pltpu.VMEM((256,), jnp.int32),
            pltpu.VMEM((256,), jnp.int32),
            pltpu.VMEM((256,), jnp.int32),
            pltpu.VMEM((4, F, _TILE), jnp.float32),
            pltpu.VMEM((2, _BLK, _TILE), jnp.float32),
            pltpu.VMEM((2, _BLK), jnp.int32),
            pltpu.SemaphoreType.DMA,
            pltpu.SemaphoreType.DMA,
            pltpu.SemaphoreType.DMA,
            pltpu.SemaphoreType.DMA,
            pltpu.SemaphoreType.DMA,
        ],
    )
    def k(uT_h, iT_h, uidx_h, iidx_h, urows_h, irows_h,
          A, UL, BL, D, segv, hist, utl, utc, slab, blk, bix,
          sem0, sem1, sem2, sem3, semS):
        wid = lax.axis_index("s") * NC + lax.axis_index("c")
        lanes = _iota()
        base_ut = wid * per + jnp.minimum(wid, extra)
        n_ut = per + (wid < extra).astype(jnp.int32)
        lo_u = base_ut * _TILE
        hi_u = (base_ut + n_ut) * _TILE
        ones = jnp.ones((_L,), jnp.int32)

        def one_table(tab_h, idx_h, rows_h):
            for g in range(256 // _L):
                hist[pl.ds(g * _L, _L)] = jnp.zeros((_L,), jnp.int32)
            for p in range(2):
                for g in range(_BLK // _L):
                    bix[p, pl.ds(g * _L, _L)] = jnp.full((_L,), B, jnp.int32)
            pltpu.sync_copy(idx_h, A.at[pl.ds(0, B)])

            def scan_g(g, cnt):
                u = A[pl.ds(g * _L, _L)]
                b = g * _L + lanes
                m = (u >= lo_u) & (u < hi_u)
                pos = cnt + plsc.cumsum(m.astype(jnp.int32)) - 1
                plsc.store_scatter(UL, [pos], u, mask=m)
                plsc.store_scatter(BL, [pos], b, mask=m)
                ut_rel = lax.shift_right_logical(u, 7) - base_ut
                plsc.addupdate_scatter(
                    hist, [jnp.where(m, ut_rel, 255)], ones, mask=m)
                return cnt + plsc.all_reduce_population_count(m)[0]

            cnt = lax.fori_loop(0, n_groups, scan_g, 0)

            def comp_g(g, c2):
                ids = g * _L + lanes
                h = hist[pl.ds(g * _L, _L)]
                m2 = (h > 0) & (ids < n_ut)
                pos = c2 + plsc.cumsum(m2.astype(jnp.int32)) - 1
                plsc.store_scatter(utl, [pos], ids, mask=m2)
                plsc.store_scatter(utc, [pos], h, mask=m2)
                return c2 + plsc.all_reduce_population_count(m2)[0]

            n_active = lax.fori_loop(0, 256 // _L, comp_g, 0)

            list_groups = lax.shift_right_logical(cnt + _L - 1, 4)
            seg_start = []
            seg_len = []
            st2 = 0
            for s in range(8):
                seg_start.append(st2)

                def bin_g(g, c3, s=s):
                    u = UL[pl.ds(g * _L, _L)]
                    b = BL[pl.ds(g * _L, _L)]
                    m = lax.shift_right_logical(u - lo_u, 12) == s
                    m = m & (g * _L + lanes < cnt)
                    pos = c3 + plsc.cumsum(m.astype(jnp.int32)) - 1
                    plsc.store_scatter(A, [pos], u, mask=m)
                    plsc.store_scatter(D, [pos], b, mask=m)
                    return c3 + plsc.all_reduce_population_count(m)[0]

                st2 = lax.fori_loop(0, list_groups, bin_g, st2)
                seg_len.append(st2 - seg_start[s])
            segs_v = jnp.zeros((_L,), jnp.int32)
            lens_v = jnp.zeros((_L,), jnp.int32)
            for s in range(8):
                segs_v = jnp.where(lanes == s, seg_start[s], segs_v)
                lens_v = jnp.where(lanes == s, seg_len[s], lens_v)
            segv[pl.ds(0, _L)] = segs_v
            segv[pl.ds(_L, _L)] = lens_v

            sems = (sem0, sem1, sem2, sem3)

            def fetch(j, ring):
                ut_rel = utl[pl.ds(j, _L)][0]
                u0 = (base_ut + ut_rel) * _TILE
                for r in range(4):
                    @pl.when(ring == r)
                    def _(r=r):
                        pltpu.async_copy(
                            tab_h.at[pl.ds(0, F), pl.ds(u0, _TILE)],
                            slab.at[r], sems[r])

            for jj in range(3):
                @pl.when(jj < n_active)
                def _(jj=jj):
                    fetch(jj, jj)

            def flush(fn):
                for p in range(2):
                    @pl.when(jnp.bitwise_and(fn, 1) == p)
                    def _(p=p):
                        pltpu.async_copy(
                            blk.at[p], rows_h.at[bix.at[p]], semS)

                @pl.when(fn >= 1)
                def _():
                    pltpu.make_async_copy(
                        rows_h.at[pl.ds(0, _BLK)], blk.at[0], semS).wait()

            def refill_bix(fn):
                for p in range(2):
                    @pl.when(jnp.bitwise_and(fn, 1) == p)
                    def _(p=p):
                        for g in range(_BLK // _L):
                            bix[p, pl.ds(g * _L, _L)] = jnp.full(
                                (_L,), B, jnp.int32)

            def ut_loop(j, carry):
                fc, fn = carry
                ring = jnp.bitwise_and(j, 3)

                @pl.when(j + 3 < n_active)
                def _():
                    fetch(j + 3, jnp.bitwise_and(j + 3, 3))

                for r in range(4):
                    @pl.when(ring == r)
                    def _(r=r):
                        pltpu.make_async_copy(
                            tab_h.at[pl.ds(0, F), pl.ds(0, _TILE)],
                            slab.at[r], sems[r]).wait()

                ut_rel = utl[pl.ds(j, _L)][0]
                k_ut = utc[pl.ds(j, _L)][0]
                s_id = lax.shift_right_logical(ut_rel, 5)
                seg0 = segv[pl.ds(s_id, _L)][0]
                slen = segv[pl.ds(s_id + _L, _L)][0]
                g0 = lax.shift_right_logical(seg0, 4)
                g1 = lax.shift_right_logical(seg0 + slen + _L - 1, 4)

                def rescan(g, st):
                    u = A[pl.ds(g * _L, _L)]
                    b = D[pl.ds(g * _L, _L)]
                    e = g * _L + lanes
                    m = (lax.shift_right_logical(u, 7) - base_ut == ut_rel)
                    m = m & (e >= seg0) & (e < seg0 + slen)
                    pos = st + plsc.cumsum(m.astype(jnp.int32)) - 1
                    plsc.store_scatter(UL, [pos], u, mask=m)
                    plsc.store_scatter(BL, [pos], b, mask=m)
                    return st + plsc.all_reduce_population_count(m)[0]

                lax.fori_loop(g0, g1, rescan, 0)

                n_chunks = lax.shift_right_logical(k_ut + _L - 1, 4)
                ringv = jnp.full((_L,), ring, jnp.int32)

                def ext(ci, c):
                    fc, fn = c
                    uvec = UL[pl.ds(ci * _L, _L)]
                    bvec = BL[pl.ds(ci * _L, _L)]
                    valid = ci * _L + lanes < k_ut
                    ui = jnp.bitwise_and(uvec, _TILE - 1)
                    par = jnp.bitwise_and(fn, 1)
                    parv = jnp.full((_L,), par, jnp.int32)
                    nsl = plsc.cumsum(valid.astype(jnp.int32))
                    slotv = fc + nsl - 1
                    plsc.store_scatter(bix, [parv, slotv], bvec, mask=valid)

                    for f in range(F):
                        fv = jnp.full((_L,), f, jnp.int32)
                        vals = plsc.load_gather(slab, [ringv, fv, ui])
                        plsc.store_scatter(
                            blk, [parv, slotv, fv], vals, mask=valid)
                    fc2 = fc + plsc.all_reduce_population_count(valid)[0]
                    do_flush = fc2 > _HI

                    @pl.when(do_flush)
                    def _():
                        flush(fn)
                        refill_bix(fn + 1)

                    fc3 = jnp.where(do_flush, 0, fc2)
                    fn2 = fn + do_flush.astype(jnp.int32)
                    return (fc3, fn2)

                return lax.fori_loop(0, n_chunks, ext, (fc, fn))

            fc, fn = lax.fori_loop(0, n_active, ut_loop, (0, 0))

            @pl.when(fc > 0)
            def _():
                flush(fn)

            fn_tot = fn + (fc > 0).astype(jnp.int32)

            @pl.when(fn_tot >= 1)
            def _():
                pltpu.make_async_copy(
                    rows_h.at[pl.ds(0, _BLK)], blk.at[0], semS).wait()

        one_table(uT_h, uidx_h, urows_h)
        one_table(iT_h, iidx_h, irows_h)

    return k


@functools.lru_cache(maxsize=None)
def _build_phase2(B, F, SB):
    info = plsc.get_sparse_core_info()
    NC, NS = info.num_cores, info.num_subcores
    NW = NC * NS
    b_per_w = B // NW
    half = b_per_w // 2
    n_chunks = b_per_w // _CHUNK

    mesh = plsc.VectorSubcoreMesh(core_axis_name="c", subcore_axis_name="s")

    @functools.partial(
        pl.kernel,
        mesh=mesh,
        out_type=jax.ShapeDtypeStruct((B,), jnp.float32),
        compiler_params=pltpu.CompilerParams(
            needs_layout_passes=False, use_tc_tiling_on_sc=False
        ),
        scratch_types=[
            pltpu.VMEM((half, _TILE), jnp.float32),
            pltpu.VMEM((half, _TILE), jnp.float32),
            pltpu.VMEM((b_per_w,), jnp.int32),
            pltpu.VMEM((b_per_w,), jnp.int32),
            pltpu.VMEM((b_per_w,), jnp.float32),
            pltpu.VMEM((b_per_w,), jnp.float32),
            pltpu.VMEM((b_per_w,), jnp.float32),
            pltpu.SemaphoreType.DMA,
        ],
    )
    def k(urows_h, irows_h, ub_h, ib_h, uidx_h, iidx_h, out_h,
          uv, iv, uidx_v, iidx_v, ubv, ibv, outv, sem):
        wid = lax.axis_index("s") * NC + lax.axis_index("c")
        lanes = _iota()
        base = wid * b_per_w
        pltpu.sync_copy(uidx_h.at[pl.ds(base, b_per_w)], uidx_v)
        pltpu.sync_copy(iidx_h.at[pl.ds(base, b_per_w)], iidx_v)
        for c in range(n_chunks):
            s = pl.ds(c * _CHUNK, _CHUNK)
            pltpu.async_copy(ub_h.at[uidx_v.at[s]], ubv.at[s], sem)
            pltpu.async_copy(ib_h.at[iidx_v.at[s]], ibv.at[s], sem)

        for h in range(2):
            pltpu.sync_copy(urows_h.at[pl.ds(base + h * half, half)], uv)
            pltpu.sync_copy(irows_h.at[pl.ds(base + h * half, half)], iv)

            def group(g, carry):
                rows = g * _L + lanes
                acc = jnp.zeros((_L,), jnp.float32)
                for f in range(F):
                    cols = jnp.bitwise_and(f + lanes, F - 1)
                    ug = plsc.load_gather(uv, [rows, cols])
                    ig = plsc.load_gather(iv, [rows, cols])
                    acc = acc + ug * ig
                outv[pl.ds(h * half + g * _L, _L)] = acc
                return carry

            lax.fori_loop(0, half // _L, group, 0)

        pltpu.make_async_copy(ub_h.at[pl.ds(0, b_per_w)], ubv, sem).wait()
        pltpu.make_async_copy(ib_h.at[pl.ds(0, b_per_w)], ibv, sem).wait()

        def addb(g, carry):
            s = pl.ds(g * _L, _L)
            outv[s] = outv[s] + ubv[s] + ibv[s]
            return carry

        lax.fori_loop(0, b_per_w // _L, addb, 0)
        pltpu.sync_copy(outv, out_h.at[pl.ds(base, b_per_w)])

    return k


def kernel(u_emb, i_emb, u_bias, i_bias, u_idx, i_idx):
    B = u_idx.shape[0]
    N, F = u_emb.shape
    u32 = u_idx.astype(jnp.int32)
    i32 = i_idx.astype(jnp.int32)
    urows, irows = _build_phase1(B, F, N)(u_emb.T, i_emb.T, u32, i32)
    return _build_phase2(B, F, B + 8)(
        urows, irows, u_bias.reshape(-1), i_bias.reshape(-1), u32, i32
    )
